# baseline (speedup 1.0000x reference)
import functools

import jax
import jax.numpy as jnp
from jax import lax
from jax.experimental import pallas as pl
from jax.experimental.pallas import tpu as pltpu
from jax.experimental.pallas import tpu_sc as plsc

_VOCAB = 1000
_HIDDEN = 128
_B = 4096
_SEQ = 20

_NC = 2
_NS = 16
_NW = _NC * _NS

_TR = _VOCAB // 8
_TRPW = 4
_NUNIT = _SEQ * _TRPW
_NTC = _B // 128


def _table_body(w_ref, we_ref, b_ref, out_ref):
    out_ref[...] = lax.dot_general(
        w_ref[...], we_ref[...],
        (((1,), (1,)), ((), ())),
        preferred_element_type=jnp.float32,
    ) + b_ref[...]


def _build_table_t(we, W, b):
    return pl.pallas_call(
        _table_body,
        out_shape=jax.ShapeDtypeStruct((_VOCAB, _VOCAB), jnp.float32),
    )(W, we, b.reshape(_VOCAB, 1))


def _gather_body(tabt_hbm, xt_hbm, out_hbm, slab, idx2, stage2,
                 ssem0, ssem1, isem0, isem1):
    cid = lax.axis_index("c")
    sid = lax.axis_index("s")
    wid = sid * _NC + cid

    lane = lax.broadcasted_iota(jnp.int32, (16,), 0) * 0
    isem = (isem0, isem1)
    ssem = (ssem0, ssem1)

    def _prefetch_idx(s, q):
        pltpu.async_copy(
            xt_hbm.at[jnp.minimum(s, _SEQ - 1)], idx2.at[q], isem[q])

    def _wait_idx(q):
        pltpu.make_async_copy(xt_hbm.at[0], idx2.at[q], isem[q]).wait()

    def _drain(p):
        for i in range(_TRPW):
            pltpu.make_async_copy(
                stage2.at[p, i], out_hbm.at[0, 0, pl.ds(0, 8)],
                ssem[p]).wait()

    _prefetch_idx(jnp.int32(0), 0)
    _prefetch_idx(jnp.int32(1), 1)

    for i in range(_TRPW):
        tr = jnp.minimum(wid + _NW * i, _TR - 1)
        pltpu.sync_copy(tabt_hbm.at[pl.ds(tr * 8, 8)], slab.at[i])

    @pl.loop(0, _NUNIT)
    def _unit(u):
        s = u // _TRPW
        qt = u % _TRPW
        p = u % 2
        q = s % 2

        @pl.when(qt == 0)
        def _wait():
            @pl.when(q == 0)
            def _():
                _wait_idx(0)

            @pl.when(q == 1)
            def _():
                _wait_idx(1)

        @pl.when(u >= 2)
        def _dr():
            @pl.when(p == 0)
            def _():
                _drain(0)

            @pl.when(p == 1)
            def _():
                _drain(1)

        @plsc.parallel_loop(0, _NTC // _TRPW, unroll=2)
        def _tc(tcl):
            for k in range(8):
                idxv = idx2[q, pl.ds((qt * 8 + tcl) * 128 + k * 16, 16)]
                for i in range(_TRPW):
                    for sl in range(8):
                        vals = plsc.load_gather(
                            slab, [lane + i, lane + sl, idxv])
                        stage2[p, i, tcl, sl, pl.ds(k * 16, 16)] = vals

        for i in range(_TRPW):
            tr = jnp.minimum(wid + _NW * i, _TR - 1)

            @pl.when(p == 0)
            def _s0():
                pltpu.async_copy(
                    stage2.at[0, i], out_hbm.at[s, tr, pl.ds(qt * 8, 8)],
                    ssem0)

            @pl.when(p == 1)
            def _s1():
                pltpu.async_copy(
                    stage2.at[1, i], out_hbm.at[s, tr, pl.ds(qt * 8, 8)],
                    ssem1)

        @pl.when(qt == 3)
        def _pf():
            @pl.when(q == 0)
            def _():
                _prefetch_idx(s + 2, 0)

            @pl.when(q == 1)
            def _():
                _prefetch_idx(s + 2, 1)

    _wait_idx(0)
    _wait_idx(1)
    _drain(0)
    _drain(1)


@functools.partial(
    pl.kernel,
    out_type=jax.ShapeDtypeStruct((_SEQ, _TR, _NTC, 8, 128), jnp.float32),
    mesh=plsc.VectorSubcoreMesh(core_axis_name="c", subcore_axis_name="s"),
    compiler_params=pltpu.CompilerParams(
        use_tc_tiling_on_sc=False, needs_layout_passes=False),
    scratch_types=[
        pltpu.VMEM((_TRPW, 8, _VOCAB), jnp.float32),
        pltpu.VMEM((2, _B), jnp.int32),
        pltpu.VMEM((2, _TRPW, _NTC // _TRPW, 8, 128), jnp.float32),
        pltpu.SemaphoreType.DMA,
        pltpu.SemaphoreType.DMA,
        pltpu.SemaphoreType.DMA,
        pltpu.SemaphoreType.DMA,
    ],
)
def _gather(tabt_hbm, xt_hbm, out_hbm, slab, idx2, stage2,
            ssem0, ssem1, isem0, isem1):
    _gather_body(tabt_hbm, xt_hbm, out_hbm, slab, idx2, stage2,
                 ssem0, ssem1, isem0, isem1)


def kernel(x, we, W, b):
    tabt = _build_table_t(we, W, b)
    xt = x.astype(jnp.int32).T
    out5 = _gather(tabt, xt)
    t = jnp.transpose(out5, (2, 4, 0, 1, 3))
    return t.reshape(_B, _SEQ, _VOCAB)


# speedup vs baseline: 4.8100x; 4.8100x over previous
import functools

import jax
import jax.numpy as jnp
from jax import lax
from jax.experimental import pallas as pl
from jax.experimental.pallas import tpu as pltpu
from jax.experimental.pallas import tpu_sc as plsc

_VOCAB = 1000
_HIDDEN = 128
_B = 4096
_SEQ = 20

_NC = 2
_NS = 16
_NW = _NC * _NS

_TR = _VOCAB // 8
_TRPW = 4
_NUNIT = _SEQ * _TRPW
_NTC = _B // 128


def _table_body(w_ref, we_ref, b_ref, out_ref):
    out_ref[...] = lax.dot_general(
        w_ref[...], we_ref[...],
        (((1,), (1,)), ((), ())),
        preferred_element_type=jnp.float32,
    ) + b_ref[...]


def _build_table_t(we, W, b):
    return pl.pallas_call(
        _table_body,
        out_shape=jax.ShapeDtypeStruct((_VOCAB, _VOCAB), jnp.float32),
    )(W, we, b.reshape(_VOCAB, 1))


def _gather_body(tabt_hbm, xt_hbm, out_hbm, slab, idx2, stage2,
                 ssem0, ssem1, isem0, isem1):
    cid = lax.axis_index("c")
    sid = lax.axis_index("s")
    wid = sid * _NC + cid

    lane = lax.broadcasted_iota(jnp.int32, (16,), 0) * 0
    isem = (isem0, isem1)
    ssem = (ssem0, ssem1)

    def _prefetch_idx(s, q):
        pltpu.async_copy(
            xt_hbm.at[jnp.minimum(s, _SEQ - 1)], idx2.at[q], isem[q])

    def _wait_idx(q):
        pltpu.make_async_copy(xt_hbm.at[0], idx2.at[q], isem[q]).wait()

    def _drain(p):
        pltpu.make_async_copy(stage2.at[p], out_hbm.at[0, 0], ssem[p]).wait()

    _prefetch_idx(jnp.int32(0), 0)
    _prefetch_idx(jnp.int32(1), 1)

    for i in range(_TRPW):
        tr = jnp.minimum(wid + _NW * i, _TR - 1)
        pltpu.sync_copy(tabt_hbm.at[pl.ds(tr * 8, 8)], slab.at[i])

    @pl.loop(0, _NUNIT)
    def _unit(u):
        s = u // _TRPW
        i = u % _TRPW
        p = u % 2
        q = s % 2

        @pl.when(i == 0)
        def _wait():
            @pl.when(q == 0)
            def _():
                _wait_idx(0)

            @pl.when(q == 1)
            def _():
                _wait_idx(1)

        @pl.when(u >= 2)
        def _dr():
            @pl.when(p == 0)
            def _():
                _drain(0)

            @pl.when(p == 1)
            def _():
                _drain(1)

        tr = jnp.minimum(wid + _NW * i, _TR - 1)
        i_splat = lane + i

        @plsc.parallel_loop(0, _NTC, unroll=2)
        def _tc(tc):
            for k in range(8):
                idxv = idx2[q, pl.ds(tc * 128 + k * 16, 16)]
                for sl in range(8):
                    vals = plsc.load_gather(
                        slab, [i_splat, lane + sl, idxv])
                    stage2[p, tc, sl, pl.ds(k * 16, 16)] = vals

        @pl.when(p == 0)
        def _s0():
            pltpu.async_copy(stage2.at[0], out_hbm.at[s, tr], ssem0)

        @pl.when(p == 1)
        def _s1():
            pltpu.async_copy(stage2.at[1], out_hbm.at[s, tr], ssem1)

        @pl.when(i == 3)
        def _pf():
            @pl.when(q == 0)
            def _():
                _prefetch_idx(s + 2, 0)

            @pl.when(q == 1)
            def _():
                _prefetch_idx(s + 2, 1)

    _wait_idx(0)
    _wait_idx(1)
    _drain(0)
    _drain(1)


@functools.partial(
    pl.kernel,
    out_type=jax.ShapeDtypeStruct((_SEQ, _TR, _NTC, 8, 128), jnp.float32),
    mesh=plsc.VectorSubcoreMesh(core_axis_name="c", subcore_axis_name="s"),
    compiler_params=pltpu.CompilerParams(
        use_tc_tiling_on_sc=False, needs_layout_passes=False),
    scratch_types=[
        pltpu.VMEM((_TRPW, 8, _VOCAB), jnp.float32),
        pltpu.VMEM((2, _B), jnp.int32),
        pltpu.VMEM((2, _NTC, 8, 128), jnp.float32),
        pltpu.SemaphoreType.DMA,
        pltpu.SemaphoreType.DMA,
        pltpu.SemaphoreType.DMA,
        pltpu.SemaphoreType.DMA,
    ],
)
def _gather(tabt_hbm, xt_hbm, out_hbm, slab, idx2, stage2,
            ssem0, ssem1, isem0, isem1):
    _gather_body(tabt_hbm, xt_hbm, out_hbm, slab, idx2, stage2,
                 ssem0, ssem1, isem0, isem1)


def kernel(x, we, W, b):
    tabt = _build_table_t(we, W, b)
    xt = x.astype(jnp.int32).T
    out5 = _gather(tabt, xt)
    t = jnp.transpose(out5, (2, 4, 0, 1, 3))
    return t.reshape(_B, _SEQ, _VOCAB)
